# Initial kernel scaffold; baseline (speedup 1.0000x reference)
import jax
import jax.numpy as jnp
from jax.experimental import pallas as pl


def kernel(x, W1, b1, W2, b2, Wp, Wq, bq, Wk, bk, Wv, bv, Wo, bo):
    raise NotImplementedError("write your pallas kernel here")


# speedup vs baseline: 1.0645x; 1.0645x over previous
import functools

import jax
import jax.numpy as jnp
import numpy as np
from jax.experimental import pallas as pl
from jax.experimental.pallas import tpu as pltpu

EMBED_DIM = 64
NUM_HEADS = 4
GRID = 10
DH = EMBED_DIM // NUM_HEADS


def _attn_body(h_ref, agg_ref, wq_ref, bq_ref, wk_ref, bk_ref, wv_ref,
               bv_ref, wo_ref, bo_ref, out_ref, wei_ref):
    head = pl.program_id(1)
    hb = h_ref[0]
    qh = hb @ wq_ref[0] + bq_ref[0]
    kh = hb @ wk_ref[0] + bk_ref[0]
    vh = hb @ wv_ref[0] + bv_ref[0]
    scores = jax.lax.dot_general(qh, kh, (((1,), (1,)), ((), ()))) * (
        1.0 / np.sqrt(DH).astype(np.float32))
    m = jnp.max(scores, axis=-1, keepdims=True)
    e = jnp.exp(scores - m)
    s = jnp.sum(e, axis=-1, keepdims=True)
    wei = e / s
    wei_ref[0, 0] = wei
    ctx = wei @ vh
    contrib = ctx @ wo_ref[0]

    @pl.when(head == 0)
    def _():
        out_ref[0] = agg_ref[0] + bo_ref[:] + contrib

    @pl.when(head > 0)
    def _():
        out_ref[0] = out_ref[0] + contrib


def _attention(h, agg, Wq, bq, Wk, bk, Wv, bv, Wo, bo):
    B, V, D = h.shape
    H = NUM_HEADS
    grid = (B, H)
    hsel = lambda r, c: pl.BlockSpec((1, r, c), lambda b, h_: (h_, 0, 0))
    out, wei = pl.pallas_call(
        _attn_body,
        grid=grid,
        in_specs=[
            pl.BlockSpec((1, V, D), lambda b, h_: (b, 0, 0)),
            pl.BlockSpec((1, V, D), lambda b, h_: (b, 0, 0)),
            hsel(D, DH), hsel(1, DH), hsel(D, DH), hsel(1, DH),
            hsel(D, DH), hsel(1, DH), hsel(DH, D),
            pl.BlockSpec((1, D), lambda b, h_: (0, 0)),
        ],
        out_specs=[
            pl.BlockSpec((1, V, D), lambda b, h_: (b, 0, 0)),
            pl.BlockSpec((1, 1, V, V), lambda b, h_: (b, h_, 0, 0)),
        ],
        out_shape=[
            jax.ShapeDtypeStruct((B, V, D), jnp.float32),
            jax.ShapeDtypeStruct((B, H, V, V), jnp.float32),
        ],
    )(h, agg,
      Wq.reshape(D, H, DH).transpose(1, 0, 2), bq.reshape(H, 1, DH),
      Wk.reshape(D, H, DH).transpose(1, 0, 2), bk.reshape(H, 1, DH),
      Wv.reshape(D, H, DH).transpose(1, 0, 2), bv.reshape(H, 1, DH),
      Wo.reshape(H, DH, D), bo.reshape(1, D))
    return out, wei


def kernel(x, W1, b1, W2, b2, Wp, Wq, bq, Wk, bk, Wv, bv, Wo, bo):
    D = EMBED_DIM
    G = GRID
    Bb, Nn, _ = x.shape
    V = G * G * G
    M = Bb * V
    ids = jnp.clip(jnp.floor(x * G).astype(jnp.int32), 0, G - 1)
    flat = ids[..., 0] * G * G + ids[..., 1] * G + ids[..., 2]
    boff = jnp.arange(Bb, dtype=jnp.int32)[:, None] * V
    vox = (flat + boff).reshape(-1)
    pts = x.reshape(-1, 3)
    ones = jnp.ones((Bb * Nn,), dtype=jnp.float32)
    counts = jax.ops.segment_sum(ones, vox, num_segments=M)
    denom = jnp.maximum(counts, 1.0)[:, None]
    centroids = jax.ops.segment_sum(pts, vox, num_segments=M) / denom
    norm_points = pts - centroids[vox]
    h1 = jax.nn.relu(norm_points @ W1 + b1)
    h1sum = jax.ops.segment_sum(h1, vox, num_segments=M)
    occ = counts[:, None] / denom
    agg = (h1sum / denom) @ W2 + occ * b2
    absm = jax.ops.segment_sum(jnp.abs(norm_points), vox, num_segments=M) / denom
    h = agg + absm @ Wp
    out, wei = _attention(h.reshape(Bb, V, D), agg.reshape(Bb, V, D),
                          Wq, bq, Wk, bk, Wv, bv, Wo, bo)
    return out.reshape(M, D), wei


# speedup vs baseline: 14.1204x; 13.2654x over previous
import functools

import jax
import jax.numpy as jnp
import numpy as np
from jax import lax
from jax.experimental import pallas as pl
from jax.experimental.pallas import tpu as pltpu
from jax.experimental.pallas import tpu_sc as plsc

EMBED_DIM = 64
NUM_HEADS = 4
GRID = 10
DH = EMBED_DIM // NUM_HEADS
B_, N_ = 4, 100000
V_ = GRID * GRID * GRID
M_ = B_ * V_
R_ = 4096

NC, NS = 2, 16
NW = NC * NS
PB = 128
NBLK = 98
CHUNK = PB * NBLK
NP = NW * CHUNK
AW = 16
FW = 80
ROWS_PER_TILE = R_ // NS


def _voxel_id(xs, ys, zs, gid):
    g = np.float32(GRID)
    ix = lax.convert_element_type(xs * g, jnp.int32)
    iy = lax.convert_element_type(ys * g, jnp.int32)
    iz = lax.convert_element_type(zs * g, jnp.int32)
    zero = jnp.zeros((16,), jnp.int32)
    nine = zero + jnp.int32(GRID - 1)
    ix = jnp.minimum(jnp.maximum(ix, zero), nine)
    iy = jnp.minimum(jnp.maximum(iy, zero), nine)
    iz = jnp.minimum(jnp.maximum(iz, zero), nine)
    v = ix * jnp.int32(GRID * GRID) + iy * jnp.int32(GRID) + iz
    b = lax.convert_element_type(
        lax.convert_element_type(gid, jnp.float32) * np.float32(1.0 / N_),
        jnp.int32)
    return v + b * jnp.int32(V_)


def _ka_body(xt_hbm, z_hbm, out_hbm, xtv, rows, idxv, stab):
    cid = lax.axis_index("c")
    sid = lax.axis_index("s")
    wid = cid * NS + sid
    r0 = sid * ROWS_PER_TILE
    pltpu.sync_copy(z_hbm.at[pl.ds(r0, ROWS_PER_TILE)],
                    stab.at[pl.ds(r0, ROWS_PER_TILE)])
    pltpu.sync_copy(xt_hbm.at[:, pl.ds(wid * CHUNK, CHUNK)], xtv)
    zv = jnp.zeros((16,), jnp.float32)

    def zr(r, c):
        rows[r] = zv
        return c
    lax.fori_loop(0, PB, zr, 0)
    plsc.subcore_barrier()
    base_gid = wid * CHUNK
    lanes = jnp.arange(16, dtype=jnp.int32)
    ones = jnp.ones((16,), jnp.float32)

    def blk(i, c0):
        def sub(j, c1):
            p0 = i * PB + j * 16
            xs = xtv[0, pl.ds(p0, 16)]
            ys = xtv[1, pl.ds(p0, 16)]
            zs = xtv[2, pl.ds(p0, 16)]
            idxv[pl.ds(j * 16, 16)] = _voxel_id(xs, ys, zs,
                                                base_gid + p0 + lanes)
            q = j * 16 + lanes
            plsc.store_scatter(rows, [q, lanes * 0], ones)
            plsc.store_scatter(rows, [q, lanes * 0 + 1], xs)
            plsc.store_scatter(rows, [q, lanes * 0 + 2], ys)
            plsc.store_scatter(rows, [q, lanes * 0 + 3], zs)
            return c1
        lax.fori_loop(0, PB // 16, sub, 0)
        pltpu.sync_copy(rows, stab.at[idxv], add=True)
        return c0
    lax.fori_loop(0, NBLK, blk, 0)
    plsc.subcore_barrier()
    pltpu.sync_copy(stab.at[pl.ds(r0, ROWS_PER_TILE)],
                    out_hbm.at[cid, pl.ds(r0, ROWS_PER_TILE)])


def _kb_body(xt_hbm, a_hbm, z_hbm, w1_hbm, b1_hbm, out_hbm,
             xtv, rows, idxv, a0v, a1v, ltab, ctab, w1v, b1v, ftab, ctab_sh):
    cid = lax.axis_index("c")
    sid = lax.axis_index("s")
    wid = cid * NS + sid
    r0 = sid * ROWS_PER_TILE
    pltpu.sync_copy(z_hbm.at[pl.ds(r0, ROWS_PER_TILE)],
                    ftab.at[pl.ds(r0, ROWS_PER_TILE)])
    pltpu.sync_copy(xt_hbm.at[:, pl.ds(wid * CHUNK, CHUNK)], xtv)
    pltpu.sync_copy(w1_hbm, w1v)
    pltpu.sync_copy(b1_hbm, b1v)
    pltpu.sync_copy(a_hbm.at[0, pl.ds(r0 * AW, ROWS_PER_TILE * AW)], a0v)
    pltpu.sync_copy(a_hbm.at[1, pl.ds(r0 * AW, ROWS_PER_TILE * AW)], a1v)
    lanes = jnp.arange(16, dtype=jnp.int32)

    def cg(g, c):
        rl = g * 16 + lanes
        fa = rl * AW
        cnt = plsc.load_gather(a0v, [fa]) + plsc.load_gather(a1v, [fa])
        inv = jnp.ones((16,), jnp.float32) / jnp.maximum(
            cnt, jnp.ones((16,), jnp.float32))
        fl = rl * 4
        for col in range(3):
            s = (plsc.load_gather(a0v, [fa + (col + 1)])
                 + plsc.load_gather(a1v, [fa + (col + 1)]))
            plsc.store_scatter(ltab, [fl + col], s * inv)
        plsc.store_scatter(ltab, [fl + 3], cnt)
        return c
    lax.fori_loop(0, ROWS_PER_TILE // 16, cg, 0)
    pltpu.sync_copy(ltab, ctab_sh.at[pl.ds(r0 * 4, ROWS_PER_TILE * 4)])
    plsc.subcore_barrier()
    pltpu.sync_copy(ctab_sh, ctab)
    w1r = [[w1v[pl.ds(j * 64 + k * 16, 16)] for k in range(4)]
           for j in range(3)]
    b1r = [b1v[pl.ds(k * 16, 16)] for k in range(4)]
    zv = jnp.zeros((16,), jnp.float32)

    def zr(r, c):
        for k in range(FW // 16):
            rows[r, pl.ds(k * 16, 16)] = zv
        return c
    lax.fori_loop(0, PB, zr, 0)
    base_gid = wid * CHUNK

    def blk(i, c0):
        def sub(j, c1):
            p0 = i * PB + j * 16
            xs = xtv[0, pl.ds(p0, 16)]
            ys = xtv[1, pl.ds(p0, 16)]
            zs = xtv[2, pl.ds(p0, 16)]
            vox = _voxel_id(xs, ys, zs, base_gid + p0 + lanes)
            idxv[pl.ds(j * 16, 16)] = vox
            fv = vox * 4
            nx = xs - plsc.load_gather(ctab, [fv])
            ny = ys - plsc.load_gather(ctab, [fv + 1])
            nz = zs - plsc.load_gather(ctab, [fv + 2])
            q = j * 16 + lanes
            plsc.store_scatter(rows, [q, lanes * 0 + 64], lax.abs(nx))
            plsc.store_scatter(rows, [q, lanes * 0 + 65], lax.abs(ny))
            plsc.store_scatter(rows, [q, lanes * 0 + 66], lax.abs(nz))
            for t in range(16):
                nxt = nx[t]
                nyt = ny[t]
                nzt = nz[t]
                for k in range(4):
                    acc = (b1r[k] + nxt * w1r[0][k] + nyt * w1r[1][k]
                           + nzt * w1r[2][k])
                    rows[j * 16 + t, pl.ds(k * 16, 16)] = jnp.maximum(
                        acc, jnp.float32(0.0))
            return c1
        lax.fori_loop(0, PB // 16, sub, 0)
        pltpu.sync_copy(rows, ftab.at[idxv], add=True)
        return c0
    lax.fori_loop(0, NBLK, blk, 0)
    plsc.subcore_barrier()
    pltpu.sync_copy(ftab.at[pl.ds(r0, ROWS_PER_TILE)],
                    out_hbm.at[cid, pl.ds(r0, ROWS_PER_TILE)])


def _scatter_stages(xp, W1, b1):
    f32 = jnp.float32
    mesh = plsc.VectorSubcoreMesh(core_axis_name="c", subcore_axis_name="s")
    ka = pl.kernel(
        _ka_body,
        out_type=jax.ShapeDtypeStruct((NC, R_, AW), f32),
        mesh=mesh,
        compiler_params=pltpu.CompilerParams(
            needs_layout_passes=False, use_tc_tiling_on_sc=False),
        scratch_types=[
            pltpu.VMEM((3, CHUNK), f32),
            pltpu.VMEM((PB, AW), f32),
            pltpu.VMEM((PB,), jnp.int32),
            pltpu.VMEM_SHARED((R_, AW), f32),
        ],
    )
    A = ka(xp, jnp.zeros((R_, AW), f32))
    kb = pl.kernel(
        _kb_body,
        out_type=jax.ShapeDtypeStruct((NC, R_, FW), f32),
        mesh=mesh,
        compiler_params=pltpu.CompilerParams(
            needs_layout_passes=False, use_tc_tiling_on_sc=False),
        scratch_types=[
            pltpu.VMEM((3, CHUNK), f32),
            pltpu.VMEM((PB, FW), f32),
            pltpu.VMEM((PB,), jnp.int32),
            pltpu.VMEM((ROWS_PER_TILE * AW,), f32),
            pltpu.VMEM((ROWS_PER_TILE * AW,), f32),
            pltpu.VMEM((ROWS_PER_TILE * 4,), f32),
            pltpu.VMEM((R_ * 4,), f32),
            pltpu.VMEM((3 * 64,), f32),
            pltpu.VMEM((64,), f32),
            pltpu.VMEM_SHARED((R_, FW), f32),
            pltpu.VMEM_SHARED((R_ * 4,), f32),
        ],
    )
    F = kb(xp, A.reshape(NC, R_ * AW), jnp.zeros((R_, FW), f32),
           W1.reshape(3 * 64), b1)
    return A, F


def _voxel_body(a0_ref, a1_ref, f0h_ref, f1h_ref, f0a_ref, f1a_ref,
                w2_ref, b2_ref, wp_ref, h_ref, agg_ref):
    cnt = a0_ref[:, 0:1] + a1_ref[:, 0:1]
    denom = jnp.maximum(cnt, 1.0)
    occ = cnt / denom
    h1m = (f0h_ref[...] + f1h_ref[...]) / denom
    am = (f0a_ref[...] + f1a_ref[...]) / denom
    agg = jnp.dot(h1m, w2_ref[...], preferred_element_type=jnp.float32)
    agg = agg + occ * b2_ref[...]
    h = agg + jnp.dot(am, wp_ref[...], preferred_element_type=jnp.float32)
    h_ref[...] = h
    agg_ref[...] = agg


def _voxel_stage(A, F, W2, b2, Wp):
    f32 = jnp.float32
    f0h, f1h = F[0, :, :64], F[1, :, :64]
    f0a, f1a = F[0, :, 64:80], F[1, :, 64:80]
    wp16 = jnp.concatenate([Wp, jnp.zeros((13, EMBED_DIM), f32)], axis=0)
    h, agg = pl.pallas_call(
        _voxel_body,
        out_shape=[
            jax.ShapeDtypeStruct((R_, EMBED_DIM), f32),
            jax.ShapeDtypeStruct((R_, EMBED_DIM), f32),
        ],
    )(A[0], A[1], f0h, f1h, f0a, f1a, W2, b2.reshape(1, EMBED_DIM), wp16)
    return h, agg


def _attn_body(h_ref, agg_ref, wq_ref, bq_ref, wk_ref, bk_ref, wv_ref,
               bv_ref, wo_ref, bo_ref, out_ref, wei_ref):
    head = pl.program_id(1)
    hb = h_ref[0]
    qh = hb @ wq_ref[0] + bq_ref[0]
    kh = hb @ wk_ref[0] + bk_ref[0]
    vh = hb @ wv_ref[0] + bv_ref[0]
    scores = jax.lax.dot_general(qh, kh, (((1,), (1,)), ((), ()))) * (
        1.0 / np.sqrt(DH).astype(np.float32))
    m = jnp.max(scores, axis=-1, keepdims=True)
    e = jnp.exp(scores - m)
    s = jnp.sum(e, axis=-1, keepdims=True)
    wei = e / s
    wei_ref[0, 0] = wei
    ctx = wei @ vh
    contrib = ctx @ wo_ref[0]

    @pl.when(head == 0)
    def _():
        out_ref[0] = agg_ref[0] + bo_ref[:] + contrib

    @pl.when(head > 0)
    def _():
        out_ref[0] = out_ref[0] + contrib


def _attention(h, agg, Wq, bq, Wk, bk, Wv, bv, Wo, bo):
    B, V, D = h.shape
    H = NUM_HEADS
    grid = (B, H)
    hsel = lambda r, c: pl.BlockSpec((1, r, c), lambda b, h_: (h_, 0, 0))
    out, wei = pl.pallas_call(
        _attn_body,
        grid=grid,
        in_specs=[
            pl.BlockSpec((1, V, D), lambda b, h_: (b, 0, 0)),
            pl.BlockSpec((1, V, D), lambda b, h_: (b, 0, 0)),
            hsel(D, DH), hsel(1, DH), hsel(D, DH), hsel(1, DH),
            hsel(D, DH), hsel(1, DH), hsel(DH, D),
            pl.BlockSpec((1, D), lambda b, h_: (0, 0)),
        ],
        out_specs=[
            pl.BlockSpec((1, V, D), lambda b, h_: (b, 0, 0)),
            pl.BlockSpec((1, 1, V, V), lambda b, h_: (b, h_, 0, 0)),
        ],
        out_shape=[
            jax.ShapeDtypeStruct((B, V, D), jnp.float32),
            jax.ShapeDtypeStruct((B, H, V, V), jnp.float32),
        ],
    )(h, agg,
      Wq.reshape(D, H, DH).transpose(1, 0, 2), bq.reshape(H, 1, DH),
      Wk.reshape(D, H, DH).transpose(1, 0, 2), bk.reshape(H, 1, DH),
      Wv.reshape(D, H, DH).transpose(1, 0, 2), bv.reshape(H, 1, DH),
      Wo.reshape(H, DH, D), bo.reshape(1, D))
    return out, wei


def kernel(x, W1, b1, W2, b2, Wp, Wq, bq, Wk, bk, Wv, bv, Wo, bo):
    D = EMBED_DIM
    Bb, Nn, _ = x.shape
    pts = x.reshape(-1, 3)
    xp = jnp.pad(pts, ((0, NP - Bb * Nn), (0, 0))).T
    A, F = _scatter_stages(xp, W1, b1)
    h, agg = _voxel_stage(A, F, W2, b2, Wp)
    out, wei = _attention(h[:M_].reshape(Bb, V_, D), agg[:M_].reshape(Bb, V_, D),
                          Wq, bq, Wk, bk, Wv, bv, Wo, bo)
    return out.reshape(M_, D), wei


# speedup vs baseline: 16.3408x; 1.1572x over previous
import functools

import jax
import jax.numpy as jnp
import numpy as np
from jax import lax
from jax.experimental import pallas as pl
from jax.experimental.pallas import tpu as pltpu
from jax.experimental.pallas import tpu_sc as plsc

EMBED_DIM = 64
NUM_HEADS = 4
GRID = 10
DH = EMBED_DIM // NUM_HEADS
B_, N_ = 4, 100000
V_ = GRID * GRID * GRID
M_ = B_ * V_
R_ = 4096

NC, NS = 2, 16
NW = NC * NS
PB = 128
NBLK = 98
CHUNK = PB * NBLK
NP = NW * CHUNK
AW = 16
FW = 80
ROWS_PER_TILE = R_ // NS


def _voxel_id(xs, ys, zs, gid):
    g = np.float32(GRID)
    ix = lax.convert_element_type(xs * g, jnp.int32)
    iy = lax.convert_element_type(ys * g, jnp.int32)
    iz = lax.convert_element_type(zs * g, jnp.int32)
    zero = jnp.zeros((16,), jnp.int32)
    nine = zero + jnp.int32(GRID - 1)
    ix = jnp.minimum(jnp.maximum(ix, zero), nine)
    iy = jnp.minimum(jnp.maximum(iy, zero), nine)
    iz = jnp.minimum(jnp.maximum(iz, zero), nine)
    v = ix * jnp.int32(GRID * GRID) + iy * jnp.int32(GRID) + iz
    b = lax.convert_element_type(
        lax.convert_element_type(gid, jnp.float32) * np.float32(1.0 / N_),
        jnp.int32)
    return v + b * jnp.int32(V_)


def _ka_body(xt_hbm, z_hbm, out_hbm, xtv, rows, idxv, stab):
    cid = lax.axis_index("c")
    sid = lax.axis_index("s")
    wid = cid * NS + sid
    r0 = sid * ROWS_PER_TILE
    pltpu.sync_copy(z_hbm.at[pl.ds(r0, ROWS_PER_TILE)],
                    stab.at[pl.ds(r0, ROWS_PER_TILE)])
    pltpu.sync_copy(xt_hbm.at[:, pl.ds(wid * CHUNK, CHUNK)], xtv)
    zv = jnp.zeros((16,), jnp.float32)

    def zr(r, c):
        rows[r] = zv
        return c
    lax.fori_loop(0, PB, zr, 0)
    plsc.subcore_barrier()
    base_gid = wid * CHUNK
    lanes = jnp.arange(16, dtype=jnp.int32)
    ones = jnp.ones((16,), jnp.float32)

    def blk(i, c0):
        def sub(j, c1):
            p0 = i * PB + j * 16
            xs = xtv[0, pl.ds(p0, 16)]
            ys = xtv[1, pl.ds(p0, 16)]
            zs = xtv[2, pl.ds(p0, 16)]
            idxv[pl.ds(j * 16, 16)] = _voxel_id(xs, ys, zs,
                                                base_gid + p0 + lanes)
            q = j * 16 + lanes
            plsc.store_scatter(rows, [q, lanes * 0], ones)
            plsc.store_scatter(rows, [q, lanes * 0 + 1], xs)
            plsc.store_scatter(rows, [q, lanes * 0 + 2], ys)
            plsc.store_scatter(rows, [q, lanes * 0 + 3], zs)
            return c1
        lax.fori_loop(0, PB // 16, sub, 0)
        pltpu.sync_copy(rows, stab.at[idxv], add=True)
        return c0
    lax.fori_loop(0, NBLK, blk, 0)
    plsc.subcore_barrier()
    pltpu.sync_copy(stab.at[pl.ds(r0, ROWS_PER_TILE)],
                    out_hbm.at[cid, pl.ds(r0, ROWS_PER_TILE)])


def _kb_body(xt_hbm, a_hbm, z_hbm, w1_hbm, b1_hbm, out_hbm,
             xtv, rows0, rows1, idxv0, idxv1, a0v, a1v, ltab, ctab, w1v, b1v,
             ftab, ctab_sh, sem0, sem1):
    cid = lax.axis_index("c")
    sid = lax.axis_index("s")
    wid = cid * NS + sid
    r0 = sid * ROWS_PER_TILE
    pltpu.sync_copy(z_hbm.at[pl.ds(r0, ROWS_PER_TILE)],
                    ftab.at[pl.ds(r0, ROWS_PER_TILE)])
    pltpu.sync_copy(xt_hbm.at[:, pl.ds(wid * CHUNK, CHUNK)], xtv)
    pltpu.sync_copy(w1_hbm, w1v)
    pltpu.sync_copy(b1_hbm, b1v)
    pltpu.sync_copy(a_hbm.at[0, pl.ds(r0 * AW, ROWS_PER_TILE * AW)], a0v)
    pltpu.sync_copy(a_hbm.at[1, pl.ds(r0 * AW, ROWS_PER_TILE * AW)], a1v)
    lanes = jnp.arange(16, dtype=jnp.int32)

    def cg(g, c):
        rl = g * 16 + lanes
        fa = rl * AW
        cnt = plsc.load_gather(a0v, [fa]) + plsc.load_gather(a1v, [fa])
        inv = jnp.ones((16,), jnp.float32) / jnp.maximum(
            cnt, jnp.ones((16,), jnp.float32))
        fl = rl * 4
        for col in range(3):
            s = (plsc.load_gather(a0v, [fa + (col + 1)])
                 + plsc.load_gather(a1v, [fa + (col + 1)]))
            plsc.store_scatter(ltab, [fl + col], s * inv)
        plsc.store_scatter(ltab, [fl + 3], cnt)
        return c
    lax.fori_loop(0, ROWS_PER_TILE // 16, cg, 0)
    pltpu.sync_copy(ltab, ctab_sh.at[pl.ds(r0 * 4, ROWS_PER_TILE * 4)])
    plsc.subcore_barrier()
    pltpu.sync_copy(ctab_sh, ctab)
    w1r = [[w1v[pl.ds(j * 64 + k * 16, 16)] for k in range(4)]
           for j in range(3)]
    b1r = [b1v[pl.ds(k * 16, 16)] for k in range(4)]
    zv = jnp.zeros((16,), jnp.float32)

    def zr(r, c):
        for k in range(FW // 16):
            rows0[r, pl.ds(k * 16, 16)] = zv
            rows1[r, pl.ds(k * 16, 16)] = zv
        return c
    lax.fori_loop(0, PB, zr, 0)
    base_gid = wid * CHUNK
    bufs = ((rows0, idxv0, sem0), (rows1, idxv1, sem1))

    def blk(i2, c0):
        for b in range(2):
            rows, idxv, sem = bufs[b]
            i = i2 * 2 + b

            @pl.when(i2 > 0)
            def _():
                pltpu.make_async_copy(rows, ftab.at[idxv], sem).wait()

            def sub(j, c1):
                p0 = i * PB + j * 16
                xs = xtv[0, pl.ds(p0, 16)]
                ys = xtv[1, pl.ds(p0, 16)]
                zs = xtv[2, pl.ds(p0, 16)]
                vox = _voxel_id(xs, ys, zs, base_gid + p0 + lanes)
                idxv[pl.ds(j * 16, 16)] = vox
                fv = vox * 4
                nx = xs - plsc.load_gather(ctab, [fv])
                ny = ys - plsc.load_gather(ctab, [fv + 1])
                nz = zs - plsc.load_gather(ctab, [fv + 2])
                q = j * 16 + lanes
                plsc.store_scatter(rows, [q, lanes * 0 + 64], lax.abs(nx))
                plsc.store_scatter(rows, [q, lanes * 0 + 65], lax.abs(ny))
                plsc.store_scatter(rows, [q, lanes * 0 + 66], lax.abs(nz))
                for t in range(16):
                    nxt = nx[t]
                    nyt = ny[t]
                    nzt = nz[t]
                    for k in range(4):
                        acc = (b1r[k] + nxt * w1r[0][k] + nyt * w1r[1][k]
                               + nzt * w1r[2][k])
                        rows[j * 16 + t, pl.ds(k * 16, 16)] = jnp.maximum(
                            acc, jnp.float32(0.0))
                return c1
            lax.fori_loop(0, PB // 16, sub, 0)
            pltpu.async_copy(rows, ftab.at[idxv], sem, add=True)
        return c0
    lax.fori_loop(0, NBLK // 2, blk, 0)
    for b in range(2):
        rows, idxv, sem = bufs[b]
        pltpu.make_async_copy(rows, ftab.at[idxv], sem).wait()
    plsc.subcore_barrier()
    pltpu.sync_copy(ftab.at[pl.ds(r0, ROWS_PER_TILE)],
                    out_hbm.at[cid, pl.ds(r0, ROWS_PER_TILE)])


def _scatter_stages(xp, W1, b1):
    f32 = jnp.float32
    mesh = plsc.VectorSubcoreMesh(core_axis_name="c", subcore_axis_name="s")
    ka = pl.kernel(
        _ka_body,
        out_type=jax.ShapeDtypeStruct((NC, R_, AW), f32),
        mesh=mesh,
        compiler_params=pltpu.CompilerParams(
            needs_layout_passes=False, use_tc_tiling_on_sc=False),
        scratch_types=[
            pltpu.VMEM((3, CHUNK), f32),
            pltpu.VMEM((PB, AW), f32),
            pltpu.VMEM((PB,), jnp.int32),
            pltpu.VMEM_SHARED((R_, AW), f32),
        ],
    )
    A = ka(xp, jnp.zeros((R_, AW), f32))
    kb = pl.kernel(
        _kb_body,
        out_type=jax.ShapeDtypeStruct((NC, R_, FW), f32),
        mesh=mesh,
        compiler_params=pltpu.CompilerParams(
            needs_layout_passes=False, use_tc_tiling_on_sc=False),
        scratch_types=[
            pltpu.VMEM((3, CHUNK), f32),
            pltpu.VMEM((PB, FW), f32),
            pltpu.VMEM((PB, FW), f32),
            pltpu.VMEM((PB,), jnp.int32),
            pltpu.VMEM((PB,), jnp.int32),
            pltpu.VMEM((ROWS_PER_TILE * AW,), f32),
            pltpu.VMEM((ROWS_PER_TILE * AW,), f32),
            pltpu.VMEM((ROWS_PER_TILE * 4,), f32),
            pltpu.VMEM((R_ * 4,), f32),
            pltpu.VMEM((3 * 64,), f32),
            pltpu.VMEM((64,), f32),
            pltpu.VMEM_SHARED((R_, FW), f32),
            pltpu.VMEM_SHARED((R_ * 4,), f32),
            pltpu.SemaphoreType.DMA,
            pltpu.SemaphoreType.DMA,
        ],
    )
    F = kb(xp, A.reshape(NC, R_ * AW), jnp.zeros((R_, FW), f32),
           W1.reshape(3 * 64), b1)
    return A, F


def _voxel_body(a0_ref, a1_ref, f0h_ref, f1h_ref, f0a_ref, f1a_ref,
                w2_ref, b2_ref, wp_ref, h_ref, agg_ref):
    cnt = a0_ref[:, 0:1] + a1_ref[:, 0:1]
    denom = jnp.maximum(cnt, 1.0)
    occ = cnt / denom
    h1m = (f0h_ref[...] + f1h_ref[...]) / denom
    am = (f0a_ref[...] + f1a_ref[...]) / denom
    agg = jnp.dot(h1m, w2_ref[...], preferred_element_type=jnp.float32)
    agg = agg + occ * b2_ref[...]
    h = agg + jnp.dot(am, wp_ref[...], preferred_element_type=jnp.float32)
    h_ref[...] = h
    agg_ref[...] = agg


def _voxel_stage(A, F, W2, b2, Wp):
    f32 = jnp.float32
    f0h, f1h = F[0, :, :64], F[1, :, :64]
    f0a, f1a = F[0, :, 64:80], F[1, :, 64:80]
    wp16 = jnp.concatenate([Wp, jnp.zeros((13, EMBED_DIM), f32)], axis=0)
    h, agg = pl.pallas_call(
        _voxel_body,
        out_shape=[
            jax.ShapeDtypeStruct((R_, EMBED_DIM), f32),
            jax.ShapeDtypeStruct((R_, EMBED_DIM), f32),
        ],
    )(A[0], A[1], f0h, f1h, f0a, f1a, W2, b2.reshape(1, EMBED_DIM), wp16)
    return h, agg


def _attn_body(h_ref, agg_ref, wq_ref, bq_ref, wk_ref, bk_ref, wv_ref,
               bv_ref, wo_ref, bo_ref, out_ref, wei_ref):
    head = pl.program_id(1)
    hb = h_ref[0]
    qh = hb @ wq_ref[0] + bq_ref[0]
    kh = hb @ wk_ref[0] + bk_ref[0]
    vh = hb @ wv_ref[0] + bv_ref[0]
    scores = jax.lax.dot_general(qh, kh, (((1,), (1,)), ((), ()))) * (
        1.0 / np.sqrt(DH).astype(np.float32))
    m = jnp.max(scores, axis=-1, keepdims=True)
    e = jnp.exp(scores - m)
    s = jnp.sum(e, axis=-1, keepdims=True)
    wei = e / s
    wei_ref[0, 0] = wei
    ctx = wei @ vh
    contrib = ctx @ wo_ref[0]

    @pl.when(head == 0)
    def _():
        out_ref[0] = agg_ref[0] + bo_ref[:] + contrib

    @pl.when(head > 0)
    def _():
        out_ref[0] = out_ref[0] + contrib


def _attention(h, agg, Wq, bq, Wk, bk, Wv, bv, Wo, bo):
    B, V, D = h.shape
    H = NUM_HEADS
    grid = (B, H)
    hsel = lambda r, c: pl.BlockSpec((1, r, c), lambda b, h_: (h_, 0, 0))
    out, wei = pl.pallas_call(
        _attn_body,
        grid=grid,
        in_specs=[
            pl.BlockSpec((1, V, D), lambda b, h_: (b, 0, 0)),
            pl.BlockSpec((1, V, D), lambda b, h_: (b, 0, 0)),
            hsel(D, DH), hsel(1, DH), hsel(D, DH), hsel(1, DH),
            hsel(D, DH), hsel(1, DH), hsel(DH, D),
            pl.BlockSpec((1, D), lambda b, h_: (0, 0)),
        ],
        out_specs=[
            pl.BlockSpec((1, V, D), lambda b, h_: (b, 0, 0)),
            pl.BlockSpec((1, 1, V, V), lambda b, h_: (b, h_, 0, 0)),
        ],
        out_shape=[
            jax.ShapeDtypeStruct((B, V, D), jnp.float32),
            jax.ShapeDtypeStruct((B, H, V, V), jnp.float32),
        ],
    )(h, agg,
      Wq.reshape(D, H, DH).transpose(1, 0, 2), bq.reshape(H, 1, DH),
      Wk.reshape(D, H, DH).transpose(1, 0, 2), bk.reshape(H, 1, DH),
      Wv.reshape(D, H, DH).transpose(1, 0, 2), bv.reshape(H, 1, DH),
      Wo.reshape(H, DH, D), bo.reshape(1, D))
    return out, wei


def kernel(x, W1, b1, W2, b2, Wp, Wq, bq, Wk, bk, Wv, bv, Wo, bo):
    D = EMBED_DIM
    Bb, Nn, _ = x.shape
    pts = x.reshape(-1, 3)
    xp = jnp.pad(pts, ((0, NP - Bb * Nn), (0, 0))).T
    A, F = _scatter_stages(xp, W1, b1)
    h, agg = _voxel_stage(A, F, W2, b2, Wp)
    out, wei = _attention(h[:M_].reshape(Bb, V_, D), agg[:M_].reshape(Bb, V_, D),
                          Wq, bq, Wk, bk, Wv, bv, Wo, bo)
    return out.reshape(M_, D), wei


# speedup vs baseline: 17.6121x; 1.0778x over previous
import jax
import jax.numpy as jnp
import numpy as np
from jax import lax
from jax.experimental import pallas as pl
from jax.experimental.pallas import tpu as pltpu
from jax.experimental.pallas import tpu_sc as plsc

EMBED_DIM = 64
NUM_HEADS = 4
GRID = 10
DH = EMBED_DIM // NUM_HEADS
B_, N_ = 4, 100000
V_ = GRID * GRID * GRID
M_ = B_ * V_
R_ = 4096

NC, NS = 2, 16
NW = NC * NS
PB = 128
NBLK = 98
CHUNK = PB * NBLK
NP = NW * CHUNK
AW = 16
FW = 80
ROWS_PER_TILE = R_ // NS


def _voxel_id(xs, ys, zs, gid):
    g = np.float32(GRID)
    ix = lax.convert_element_type(xs * g, jnp.int32)
    iy = lax.convert_element_type(ys * g, jnp.int32)
    iz = lax.convert_element_type(zs * g, jnp.int32)
    zero = jnp.zeros((16,), jnp.int32)
    nine = zero + jnp.int32(GRID - 1)
    ix = jnp.minimum(jnp.maximum(ix, zero), nine)
    iy = jnp.minimum(jnp.maximum(iy, zero), nine)
    iz = jnp.minimum(jnp.maximum(iz, zero), nine)
    v = ix * jnp.int32(GRID * GRID) + iy * jnp.int32(GRID) + iz
    b = lax.convert_element_type(
        lax.convert_element_type(gid, jnp.float32) * np.float32(1.0 / N_),
        jnp.int32)
    return v + b * jnp.int32(V_)


def _ka_body(xt_hbm, z_hbm, out_hbm, xtv, rows0, rows1, idxv0, idxv1,
             stab, sem0, sem1):
    cid = lax.axis_index("c")
    sid = lax.axis_index("s")
    wid = cid * NS + sid
    r0 = sid * ROWS_PER_TILE
    pltpu.sync_copy(z_hbm.at[pl.ds(r0, ROWS_PER_TILE)],
                    stab.at[pl.ds(r0, ROWS_PER_TILE)])
    pltpu.sync_copy(xt_hbm.at[:, pl.ds(wid * CHUNK, CHUNK)], xtv)
    zv = jnp.zeros((16,), jnp.float32)

    def zr(r, c):
        rows0[r] = zv
        rows1[r] = zv
        return c
    lax.fori_loop(0, PB, zr, 0)
    plsc.subcore_barrier()
    base_gid = wid * CHUNK
    lanes = jnp.arange(16, dtype=jnp.int32)
    ones = jnp.ones((16,), jnp.float32)
    bufs = ((rows0, idxv0, sem0), (rows1, idxv1, sem1))

    def blk(i2, c0):
        for b in range(2):
            rows, idxv, sem = bufs[b]
            i = i2 * 2 + b

            @pl.when(i2 > 0)
            def _():
                pltpu.make_async_copy(rows, stab.at[idxv], sem).wait()

            def sub(j, c1):
                p0 = i * PB + j * 16
                xs = xtv[0, pl.ds(p0, 16)]
                ys = xtv[1, pl.ds(p0, 16)]
                zs = xtv[2, pl.ds(p0, 16)]
                idxv[pl.ds(j * 16, 16)] = _voxel_id(xs, ys, zs,
                                                    base_gid + p0 + lanes)
                q = j * 16 + lanes
                plsc.store_scatter(rows, [q, lanes * 0], ones)
                plsc.store_scatter(rows, [q, lanes * 0 + 1], xs)
                plsc.store_scatter(rows, [q, lanes * 0 + 2], ys)
                plsc.store_scatter(rows, [q, lanes * 0 + 3], zs)
                return c1
            lax.fori_loop(0, PB // 16, sub, 0)
            pltpu.async_copy(rows, stab.at[idxv], sem, add=True)
        return c0
    lax.fori_loop(0, NBLK // 2, blk, 0)
    for b in range(2):
        rows, idxv, sem = bufs[b]
        pltpu.make_async_copy(rows, stab.at[idxv], sem).wait()
    plsc.subcore_barrier()
    pltpu.sync_copy(stab.at[pl.ds(r0, ROWS_PER_TILE)],
                    out_hbm.at[cid, pl.ds(r0, ROWS_PER_TILE)])


def _kb_body(xt_hbm, a_hbm, z_hbm, w1_hbm, b1_hbm, out_hbm,
             xtv, rows0, rows1, idxv0, idxv1, a0v, a1v, ltab, ctab, w1v, b1v,
             ftab, ctab_sh, sem0, sem1):
    cid = lax.axis_index("c")
    sid = lax.axis_index("s")
    wid = cid * NS + sid
    r0 = sid * ROWS_PER_TILE
    pltpu.sync_copy(z_hbm.at[pl.ds(r0, ROWS_PER_TILE)],
                    ftab.at[pl.ds(r0, ROWS_PER_TILE)])
    pltpu.sync_copy(xt_hbm.at[:, pl.ds(wid * CHUNK, CHUNK)], xtv)
    pltpu.sync_copy(w1_hbm, w1v)
    pltpu.sync_copy(b1_hbm, b1v)
    pltpu.sync_copy(a_hbm.at[0, pl.ds(r0 * AW, ROWS_PER_TILE * AW)], a0v)
    pltpu.sync_copy(a_hbm.at[1, pl.ds(r0 * AW, ROWS_PER_TILE * AW)], a1v)
    lanes = jnp.arange(16, dtype=jnp.int32)

    def cg(g, c):
        rl = g * 16 + lanes
        fa = rl * AW
        cnt = plsc.load_gather(a0v, [fa]) + plsc.load_gather(a1v, [fa])
        inv = jnp.ones((16,), jnp.float32) / jnp.maximum(
            cnt, jnp.ones((16,), jnp.float32))
        fl = rl * 4
        for col in range(3):
            s = (plsc.load_gather(a0v, [fa + (col + 1)])
                 + plsc.load_gather(a1v, [fa + (col + 1)]))
            plsc.store_scatter(ltab, [fl + col], s * inv)
        plsc.store_scatter(ltab, [fl + 3], cnt)
        return c
    lax.fori_loop(0, ROWS_PER_TILE // 16, cg, 0)
    pltpu.sync_copy(ltab, ctab_sh.at[pl.ds(r0 * 4, ROWS_PER_TILE * 4)])
    plsc.subcore_barrier()
    pltpu.sync_copy(ctab_sh, ctab)
    w1r = [[w1v[pl.ds(j * 64 + k * 16, 16)] for k in range(4)]
           for j in range(3)]
    b1r = [b1v[pl.ds(k * 16, 16)] for k in range(4)]
    zv = jnp.zeros((16,), jnp.float32)

    def zr(r, c):
        for k in range(FW // 16):
            rows0[r, pl.ds(k * 16, 16)] = zv
            rows1[r, pl.ds(k * 16, 16)] = zv
        return c
    lax.fori_loop(0, PB, zr, 0)
    base_gid = wid * CHUNK
    bufs = ((rows0, idxv0, sem0), (rows1, idxv1, sem1))

    def blk(i2, c0):
        for b in range(2):
            rows, idxv, sem = bufs[b]
            i = i2 * 2 + b

            @pl.when(i2 > 0)
            def _():
                pltpu.make_async_copy(rows, ftab.at[idxv], sem).wait()

            def sub(j, c1):
                p0 = i * PB + j * 16
                xs = xtv[0, pl.ds(p0, 16)]
                ys = xtv[1, pl.ds(p0, 16)]
                zs = xtv[2, pl.ds(p0, 16)]
                vox = _voxel_id(xs, ys, zs, base_gid + p0 + lanes)
                idxv[pl.ds(j * 16, 16)] = vox
                fv = vox * 4
                nx = xs - plsc.load_gather(ctab, [fv])
                ny = ys - plsc.load_gather(ctab, [fv + 1])
                nz = zs - plsc.load_gather(ctab, [fv + 2])
                q = j * 16 + lanes
                plsc.store_scatter(rows, [q, lanes * 0 + 64], lax.abs(nx))
                plsc.store_scatter(rows, [q, lanes * 0 + 65], lax.abs(ny))
                plsc.store_scatter(rows, [q, lanes * 0 + 66], lax.abs(nz))
                for t in range(16):
                    nxt = nx[t]
                    nyt = ny[t]
                    nzt = nz[t]
                    for k in range(4):
                        acc = (b1r[k] + nxt * w1r[0][k] + nyt * w1r[1][k]
                               + nzt * w1r[2][k])
                        rows[j * 16 + t, pl.ds(k * 16, 16)] = jnp.maximum(
                            acc, jnp.float32(0.0))
                return c1
            lax.fori_loop(0, PB // 16, sub, 0)
            pltpu.async_copy(rows, ftab.at[idxv], sem, add=True)
        return c0
    lax.fori_loop(0, NBLK // 2, blk, 0)
    for b in range(2):
        rows, idxv, sem = bufs[b]
        pltpu.make_async_copy(rows, ftab.at[idxv], sem).wait()
    plsc.subcore_barrier()
    pltpu.sync_copy(ftab.at[pl.ds(r0, ROWS_PER_TILE)],
                    out_hbm.at[cid, pl.ds(r0, ROWS_PER_TILE)])


def _scatter_stages(xp, W1, b1):
    f32 = jnp.float32
    mesh = plsc.VectorSubcoreMesh(core_axis_name="c", subcore_axis_name="s")
    ka = pl.kernel(
        _ka_body,
        out_type=jax.ShapeDtypeStruct((NC, R_, AW), f32),
        mesh=mesh,
        compiler_params=pltpu.CompilerParams(
            needs_layout_passes=False, use_tc_tiling_on_sc=False),
        scratch_types=[
            pltpu.VMEM((3, CHUNK), f32),
            pltpu.VMEM((PB, AW), f32),
            pltpu.VMEM((PB, AW), f32),
            pltpu.VMEM((PB,), jnp.int32),
            pltpu.VMEM((PB,), jnp.int32),
            pltpu.VMEM_SHARED((R_, AW), f32),
            pltpu.SemaphoreType.DMA,
            pltpu.SemaphoreType.DMA,
        ],
    )
    A = ka(xp, jnp.zeros((R_, AW), f32))
    kb = pl.kernel(
        _kb_body,
        out_type=jax.ShapeDtypeStruct((NC, R_, FW), f32),
        mesh=mesh,
        compiler_params=pltpu.CompilerParams(
            needs_layout_passes=False, use_tc_tiling_on_sc=False),
        scratch_types=[
            pltpu.VMEM((3, CHUNK), f32),
            pltpu.VMEM((PB, FW), f32),
            pltpu.VMEM((PB, FW), f32),
            pltpu.VMEM((PB,), jnp.int32),
            pltpu.VMEM((PB,), jnp.int32),
            pltpu.VMEM((ROWS_PER_TILE * AW,), f32),
            pltpu.VMEM((ROWS_PER_TILE * AW,), f32),
            pltpu.VMEM((ROWS_PER_TILE * 4,), f32),
            pltpu.VMEM((R_ * 4,), f32),
            pltpu.VMEM((3 * 64,), f32),
            pltpu.VMEM((64,), f32),
            pltpu.VMEM_SHARED((R_, FW), f32),
            pltpu.VMEM_SHARED((R_ * 4,), f32),
            pltpu.SemaphoreType.DMA,
            pltpu.SemaphoreType.DMA,
        ],
    )
    F = kb(xp, A.reshape(NC, R_ * AW), jnp.zeros((R_, FW), f32),
           W1.reshape(3 * 64), b1)
    return A, F


def _attn_body(a0_ref, a1_ref, f0h_ref, f1h_ref, f0a_ref, f1a_ref,
               w2_ref, b2_ref, wp_ref, wq_ref, bq_ref, wk_ref, bk_ref,
               wv_ref, bv_ref, wo_ref, bo_ref, out_ref, wei_ref,
               h_scr, agg_scr):
    bidx = pl.program_id(0)
    head = pl.program_id(1)

    @pl.when(jnp.logical_and(bidx == 0, head == 0))
    def _():
        cnt = a0_ref[:, 0:1] + a1_ref[:, 0:1]
        denom = jnp.maximum(cnt, 1.0)
        occ = cnt / denom
        h1m = (f0h_ref[...] + f1h_ref[...]) / denom
        am = (f0a_ref[...] + f1a_ref[...]) / denom
        agg = jnp.dot(h1m, w2_ref[...], preferred_element_type=jnp.float32)
        agg = agg + occ * b2_ref[...]
        agg_scr[...] = agg
        h_scr[...] = agg + jnp.dot(am, wp_ref[...],
                                   preferred_element_type=jnp.float32)

    hb = h_scr[pl.ds(bidx * V_, V_), :]
    qh = hb @ wq_ref[0] + bq_ref[0]
    kh = hb @ wk_ref[0] + bk_ref[0]
    vh = hb @ wv_ref[0] + bv_ref[0]
    scores = jax.lax.dot_general(qh, kh, (((1,), (1,)), ((), ()))) * (
        1.0 / np.sqrt(DH).astype(np.float32))
    m = jnp.max(scores, axis=-1, keepdims=True)
    e = jnp.exp(scores - m)
    s = jnp.sum(e, axis=-1, keepdims=True)
    wei = e / s
    wei_ref[0, 0] = wei
    ctx = wei @ vh
    contrib = ctx @ wo_ref[0]

    @pl.when(head == 0)
    def _():
        out_ref[0] = agg_scr[pl.ds(bidx * V_, V_), :] + bo_ref[:] + contrib

    @pl.when(head > 0)
    def _():
        out_ref[0] = out_ref[0] + contrib


def _attention(A, F, W2, b2, Wp, Wq, bq, Wk, bk, Wv, bv, Wo, bo):
    B, V, D = B_, V_, EMBED_DIM
    H = NUM_HEADS
    f32 = jnp.float32
    f0h, f1h = F[0, :, :64], F[1, :, :64]
    f0a, f1a = F[0, :, 64:80], F[1, :, 64:80]
    wp16 = jnp.concatenate([Wp, jnp.zeros((13, D), f32)], axis=0)
    grid = (B, H)
    full = lambda r, c: pl.BlockSpec((r, c), lambda b, h_: (0, 0))
    hsel = lambda r, c: pl.BlockSpec((1, r, c), lambda b, h_: (h_, 0, 0))
    out, wei = pl.pallas_call(
        _attn_body,
        grid=grid,
        in_specs=[
            full(R_, AW), full(R_, AW), full(R_, 64), full(R_, 64),
            full(R_, 16), full(R_, 16), full(D, D), full(1, D), full(16, D),
            hsel(D, DH), hsel(1, DH), hsel(D, DH), hsel(1, DH),
            hsel(D, DH), hsel(1, DH), hsel(DH, D),
            pl.BlockSpec((1, D), lambda b, h_: (0, 0)),
        ],
        out_specs=[
            pl.BlockSpec((1, V, D), lambda b, h_: (b, 0, 0)),
            pl.BlockSpec((1, 1, V, V), lambda b, h_: (b, h_, 0, 0)),
        ],
        out_shape=[
            jax.ShapeDtypeStruct((B, V, D), jnp.float32),
            jax.ShapeDtypeStruct((B, H, V, V), jnp.float32),
        ],
        scratch_shapes=[
            pltpu.VMEM((R_, D), f32),
            pltpu.VMEM((R_, D), f32),
        ],
    )(A[0], A[1], f0h, f1h, f0a, f1a, W2, b2.reshape(1, D), wp16,
      Wq.reshape(D, H, DH).transpose(1, 0, 2), bq.reshape(H, 1, DH),
      Wk.reshape(D, H, DH).transpose(1, 0, 2), bk.reshape(H, 1, DH),
      Wv.reshape(D, H, DH).transpose(1, 0, 2), bv.reshape(H, 1, DH),
      Wo.reshape(H, DH, D), bo.reshape(1, D))
    return out, wei


def kernel(x, W1, b1, W2, b2, Wp, Wq, bq, Wk, bk, Wv, bv, Wo, bo):
    D = EMBED_DIM
    Bb, Nn, _ = x.shape
    pts = x.reshape(-1, 3)
    xp = jnp.pad(pts, ((0, NP - Bb * Nn), (0, 0))).T
    A, F = _scatter_stages(xp, W1, b1)
    out, wei = _attention(A, F, W2, b2, Wp, Wq, bq, Wk, bk, Wv, bv, Wo, bo)
    return out.reshape(M_, D), wei
